# baseline (speedup 1.0000x reference)
import functools
import math

import jax
import jax.numpy as jnp
from jax import lax
from jax.experimental import pallas as pl
from jax.experimental.pallas import tpu as pltpu
from jax.experimental.pallas import tpu_sc as plsc

_BD = 128
_VB = 256
_N = 256
_TOPK = 8
_T = 256
_NW = 32
_TW = 128


def _body_a(hs_ref, sk_ref, rel_ref, wq_ref, s3_ref, sk_s, rel_s, wqT_s):
    i = pl.program_id(0)

    @pl.when(i == 0)
    def _prep():
        sk_s[...] = sk_ref[...].astype(jnp.bfloat16)
        rel_s[...] = jnp.log(jnp.clip(rel_ref[...], 1e-10, None))
        wqT_s[...] = wq_ref[...].T.astype(jnp.bfloat16)

    scale = 1.0 / math.sqrt(_BD)
    hs = hs_ref[...].astype(jnp.bfloat16)
    q = jnp.dot(hs, wqT_s[...], preferred_element_type=jnp.float32)
    qkT = lax.dot_general(sk_s[...], q.astype(jnp.bfloat16),
                          (((1,), (1,)), ((), ())),
                          preferred_element_type=jnp.float32) * scale
    scoresT = qkT + rel_s[...]
    s3_ref[0] = scoresT[:, :_TW]
    s3_ref[1] = scoresT[:, _TW:]


def _sc_select(s3_hbm, m3_hbm, m_v, sem):
    wid = lax.axis_index("s") * 2 + lax.axis_index("c")
    pltpu.sync_copy(s3_hbm.at[wid], m_v)
    neg = jnp.full((16,), -1e30, jnp.float32)
    for _ in range(_TOPK):
        def red(j, accs):
            return tuple(
                jnp.maximum(a, m_v[j, pl.ds(g * 16, 16)])
                for g, a in enumerate(accs))
        vms = lax.fori_loop(
            0, _N, red,
            tuple(jnp.full((16,), -3e38, jnp.float32) for _ in range(8)))

        def mask(j, c):
            for g in range(8):
                row = m_v[j, pl.ds(g * 16, 16)]
                m_v[j, pl.ds(g * 16, 16)] = jnp.where(row == vms[g], neg, row)
            return c
        lax.fori_loop(0, _N, mask, 0)
    pltpu.sync_copy(m_v, m3_hbm.at[wid])


def _body_c(prim_ref, s3_ref, m3_ref, sv_ref, rel_ref, wd_ref, wu_ref,
            out_ref, svd_s, rel_s, wuT_s):
    i = pl.program_id(0)

    @pl.when(i == 0)
    def _prep():
        rel_s[...] = jnp.log(jnp.clip(rel_ref[...], 1e-10, None))
        wuT_s[...] = wu_ref[...].T.astype(jnp.bfloat16)
        svd = lax.dot_general(sv_ref[...].astype(jnp.bfloat16),
                              wd_ref[...].astype(jnp.bfloat16),
                              (((1,), (1,)), ((), ())),
                              preferred_element_type=jnp.float32)
        svd_s[...] = svd.astype(jnp.bfloat16)

    scoresT = jnp.concatenate([s3_ref[0], s3_ref[1]], axis=1)
    mT = jnp.concatenate([m3_ref[0], m3_ref[1]], axis=1)
    sel = mT < -1e29
    gate_row = jax.nn.sigmoid(jnp.max(scoresT, axis=0, keepdims=True))
    qkT = scoresT - rel_s[...]
    logitsT = jnp.where(sel, qkT, -1e30)
    lmax = jnp.max(logitsT, axis=0, keepdims=True)
    p = jnp.exp(logitsT - lmax) * sel.astype(jnp.float32)
    w = p / jnp.sum(p, axis=0, keepdims=True)

    a = lax.dot_general(w.astype(jnp.bfloat16), svd_s[...],
                        (((0,), (0,)), ((), ())),
                        preferred_element_type=jnp.float32)
    g = jax.nn.gelu(a)
    u = jnp.dot(g.astype(jnp.bfloat16), wuT_s[...],
                preferred_element_type=jnp.float32)
    gate = gate_row.reshape(_T, 1)
    out_ref[...] = prim_ref[...] + gate * u


def kernel(hidden_states, primary_attention_output, slot_keys, slot_values,
           reliability, Wq, Wd, Wu):
    B, S, H = hidden_states.shape
    M = B * S
    hs = hidden_states.reshape(M, H)
    prim = primary_attention_output.reshape(M, H)
    rel = reliability.reshape(_N, 1)

    const = lambda i: (0, 0)
    const3 = lambda i: (i, 0, 0)

    scores3 = pl.pallas_call(
        _body_a,
        grid=(M // _T,),
        in_specs=[
            pl.BlockSpec((_T, H), lambda i: (i, 0)),
            pl.BlockSpec((_N, _BD), const),
            pl.BlockSpec((_N, 1), const),
            pl.BlockSpec((_BD, H), const),
        ],
        out_specs=pl.BlockSpec((2, _N, _TW), const3),
        out_shape=jax.ShapeDtypeStruct((_NW, _N, _TW), jnp.float32),
        scratch_shapes=[
            pltpu.VMEM((_N, _BD), jnp.bfloat16),
            pltpu.VMEM((_N, 1), jnp.float32),
            pltpu.VMEM((H, _BD), jnp.bfloat16),
        ],
        compiler_params=pltpu.CompilerParams(
            dimension_semantics=("arbitrary",),
        ),
    )(hs, slot_keys, rel, Wq)

    mesh = plsc.VectorSubcoreMesh(core_axis_name="c", subcore_axis_name="s",
                                  num_cores=2, num_subcores=16)
    masked3 = pl.kernel(
        _sc_select,
        out_type=jax.ShapeDtypeStruct((_NW, _N, _TW), jnp.float32),
        mesh=mesh,
        scratch_types=[
            pltpu.VMEM((_N, _TW), jnp.float32),
            pltpu.SemaphoreType.DMA,
        ],
    )(scores3)

    out = pl.pallas_call(
        _body_c,
        grid=(M // _T,),
        in_specs=[
            pl.BlockSpec((_T, H), lambda i: (i, 0)),
            pl.BlockSpec((2, _N, _TW), const3),
            pl.BlockSpec((2, _N, _TW), const3),
            pl.BlockSpec((_N, H), const),
            pl.BlockSpec((_N, 1), const),
            pl.BlockSpec((_VB, H), const),
            pl.BlockSpec((H, _VB), const),
        ],
        out_specs=pl.BlockSpec((_T, H), lambda i: (i, 0)),
        out_shape=jax.ShapeDtypeStruct((M, H), jnp.float32),
        scratch_shapes=[
            pltpu.VMEM((_N, _VB), jnp.bfloat16),
            pltpu.VMEM((_N, 1), jnp.float32),
            pltpu.VMEM((_VB, H), jnp.bfloat16),
        ],
        compiler_params=pltpu.CompilerParams(
            dimension_semantics=("arbitrary",),
        ),
    )(prim, scores3, masked3, slot_values, rel, Wd, Wu)
    return out.reshape(B, S, H)


# speedup vs baseline: 1.9850x; 1.9850x over previous
import math

import jax
import jax.numpy as jnp
from jax import lax
from jax.experimental import pallas as pl
from jax.experimental.pallas import tpu as pltpu

_BD = 128
_VB = 256
_N = 256
_TOPK = 8
_T = 256


def _body(hs_ref, prim_ref, sk_ref, sv_ref, rel_ref, wq_ref, wd_ref, wu_ref,
          out_ref, sk_s, svd_s, rel_s, wqT_s, wuT_s):
    i = pl.program_id(0)

    @pl.when(i == 0)
    def _prep():
        sk_s[...] = sk_ref[...].astype(jnp.bfloat16)
        rel_s[...] = jnp.log(jnp.clip(rel_ref[...], 1e-10, None))
        wqT_s[...] = wq_ref[...].T.astype(jnp.bfloat16)
        wuT_s[...] = wu_ref[...].T.astype(jnp.bfloat16)
        svd = lax.dot_general(sv_ref[...].astype(jnp.bfloat16),
                              wd_ref[...].astype(jnp.bfloat16),
                              (((1,), (1,)), ((), ())),
                              preferred_element_type=jnp.float32)
        svd_s[...] = svd.astype(jnp.bfloat16)

    scale = 1.0 / math.sqrt(_BD)
    hs = hs_ref[...].astype(jnp.bfloat16)
    q = jnp.dot(hs, wqT_s[...], preferred_element_type=jnp.float32)
    qkT = lax.dot_general(sk_s[...], q.astype(jnp.bfloat16),
                          (((1,), (1,)), ((), ())),
                          preferred_element_type=jnp.float32) * scale
    scoresT = qkT + rel_s[...]

    ids = lax.broadcasted_iota(jnp.int32, scoresT.shape, 0)
    m = scoresT
    gate_row = None
    for k in range(_TOPK):
        cmax = jnp.max(m, axis=0, keepdims=True)
        if k == 0:
            gate_row = jax.nn.sigmoid(cmax)
        idx = jnp.min(jnp.where(m == cmax, ids, _N), axis=0, keepdims=True)
        m = jnp.where(ids == idx, -1e30, m)
    sel = m < -1e29

    logitsT = jnp.where(sel, qkT, -1e30)
    lmax = jnp.max(logitsT, axis=0, keepdims=True)
    p = jnp.exp(logitsT - lmax) * sel.astype(jnp.float32)
    w = p / jnp.sum(p, axis=0, keepdims=True)

    a = lax.dot_general(w.astype(jnp.bfloat16), svd_s[...],
                        (((0,), (0,)), ((), ())),
                        preferred_element_type=jnp.float32)
    g = jax.nn.gelu(a)
    u = jnp.dot(g.astype(jnp.bfloat16), wuT_s[...],
                preferred_element_type=jnp.float32)
    gate = gate_row.reshape(_T, 1)
    out_ref[...] = prim_ref[...] + gate * u


def kernel(hidden_states, primary_attention_output, slot_keys, slot_values,
           reliability, Wq, Wd, Wu):
    B, S, H = hidden_states.shape
    M = B * S
    hs = hidden_states.reshape(M, H)
    prim = primary_attention_output.reshape(M, H)
    rel = reliability.reshape(_N, 1)

    const = lambda i: (0, 0)
    out = pl.pallas_call(
        _body,
        grid=(M // _T,),
        in_specs=[
            pl.BlockSpec((_T, H), lambda i: (i, 0)),
            pl.BlockSpec((_T, H), lambda i: (i, 0)),
            pl.BlockSpec((_N, _BD), const),
            pl.BlockSpec((_N, H), const),
            pl.BlockSpec((_N, 1), const),
            pl.BlockSpec((_BD, H), const),
            pl.BlockSpec((_VB, H), const),
            pl.BlockSpec((H, _VB), const),
        ],
        out_specs=pl.BlockSpec((_T, H), lambda i: (i, 0)),
        out_shape=jax.ShapeDtypeStruct((M, H), jnp.float32),
        scratch_shapes=[
            pltpu.VMEM((_N, _BD), jnp.bfloat16),
            pltpu.VMEM((_N, _VB), jnp.bfloat16),
            pltpu.VMEM((_N, 1), jnp.float32),
            pltpu.VMEM((H, _BD), jnp.bfloat16),
            pltpu.VMEM((_VB, H), jnp.bfloat16),
        ],
        compiler_params=pltpu.CompilerParams(
            dimension_semantics=("arbitrary",),
        ),
    )(hs, prim, slot_keys, slot_values, rel, Wq, Wd, Wu)
    return out.reshape(B, S, H)
